# baseline (speedup 1.0000x reference)
import jax
import jax.numpy as jnp
from jax import lax
from jax.experimental import pallas as pl
from jax.experimental.pallas import tpu as pltpu
from jax.experimental.pallas import tpu_sc as plsc

VOCAB = 1000000
EMBED = 32
NUM_WORDS = 5
HIDDEN = 256
OUT = 64
BATCH = 16384

ROWS = BATCH * NUM_WORDS
CHUNK = 128
NC = 2
NS = 16
NW = NC * NS
RPW = ROWS // NW
CPW = RPW // CHUNK
NBUF = 2

GPR = 8
PAIRS = EMBED // 2
GROUP_W = GPR * PAIRS
GROUPS = VOCAB // GPR
OUT_ROWS = ROWS * PAIRS // 128

BLKV = 512
ORB = BLKV // GPR
NFULL = VOCAB // BLKV
TAILV = VOCAB - NFULL * BLKV
BPW = NFULL // NW

_IOTA = None


def _k1_body(tt_hbm, tail_hbm, out_hbm, buf0, buf1, ob0, ob1, tob,
             sem0, sem1, osem0, osem1):
    wid = lax.axis_index("s") * NC + lax.axis_index("c")
    iota = lax.iota(jnp.int32, 16)
    nb = jnp.where(wid == 0, BPW + 1, BPW)

    def blk_of(i):
        return jnp.where(i < BPW, wid * BPW + i, NFULL - 1)

    def fire(blk, buf, sem):
        v0 = blk * BLKV
        for te in range(4):
            pltpu.async_copy(
                tt_hbm.at[pl.ds(te * 8, 8), pl.ds(v0, BLKV)],
                buf.at[pl.ds(te * 8, 8), :], sem)

    def wait_buf(buf, sem):
        pltpu.make_async_copy(tt_hbm.at[:, pl.ds(0, BLKV)], buf, sem).wait()

    def extract(buf, ob, ncol):
        for pw in range(PAIRS):
            lo = jnp.full((16,), 2 * pw, jnp.int32)
            hi = jnp.full((16,), 2 * pw + 1, jnp.int32)

            def g_body(g, carry):
                c16 = g * 16 + iota
                va = plsc.load_gather(buf, [lo, c16])
                vb = plsc.load_gather(buf, [hi, c16])
                packed = plsc.pack(va, vb,
                                   format=plsc.PackFormat.INTERLEAVED)
                u = plsc.bitcast(packed, jnp.int32)
                rvec = lax.shift_right_logical(c16, 3)
                colv = lax.shift_left(jnp.bitwise_and(c16, 7), 4) + pw
                plsc.store_scatter(ob, [rvec, colv], u)
                return carry

            lax.fori_loop(0, ncol // 16, g_body, 0)

    fire(blk_of(0), buf0, sem0)

    @pl.when(nb > 1)
    def _():
        fire(blk_of(1), buf1, sem1)

    def do_block(i, buf, ob, sem, osem):
        blk = blk_of(i)
        wait_buf(buf, sem)

        @pl.when(i >= 2)
        def _():
            pltpu.make_async_copy(out_hbm.at[pl.ds(0, ORB)], ob, osem).wait()
        extract(buf, ob, BLKV)

        @pl.when(i + 2 < nb)
        def _():
            fire(blk_of(i + 2), buf, sem)
        pltpu.async_copy(ob, out_hbm.at[pl.ds(blk * ORB, ORB)], osem)

    def step(i, carry):
        @pl.when(lax.rem(i, 2) == 0)
        def _():
            do_block(i, buf0, ob0, sem0, osem0)

        @pl.when(lax.rem(i, 2) == 1)
        def _():
            do_block(i, buf1, ob1, sem1, osem1)
        return carry

    lax.fori_loop(0, nb, step, 0)

    @pl.when(nb >= 1)
    def _():
        pltpu.make_async_copy(out_hbm.at[pl.ds(0, ORB)], ob0, osem0).wait()

    @pl.when(nb >= 2)
    def _():
        pltpu.make_async_copy(out_hbm.at[pl.ds(0, ORB)], ob1, osem1).wait()

    @pl.when(wid == 1)
    def _():
        pltpu.sync_copy(tail_hbm, tob)
        pltpu.sync_copy(tob, out_hbm.at[pl.ds(NFULL * ORB, TAILV // GPR)])


_k1 = pl.kernel(
    _k1_body,
    out_type=jax.ShapeDtypeStruct((GROUPS, GROUP_W), jnp.int32),
    mesh=plsc.VectorSubcoreMesh(core_axis_name="c", subcore_axis_name="s"),
    scratch_types=[
        pltpu.VMEM((EMBED, BLKV), jnp.float32),
        pltpu.VMEM((EMBED, BLKV), jnp.float32),
        pltpu.VMEM((ORB, GROUP_W), jnp.int32),
        pltpu.VMEM((ORB, GROUP_W), jnp.int32),
        pltpu.VMEM((TAILV // GPR, GROUP_W), jnp.int32),
        pltpu.SemaphoreType.DMA,
        pltpu.SemaphoreType.DMA,
        pltpu.SemaphoreType.DMA,
        pltpu.SemaphoreType.DMA,
    ],
    compiler_params=pltpu.CompilerParams(
        use_tc_tiling_on_sc=True, needs_layout_passes=False
    ),
)


def _k2_body(tpack_hbm, idx_hbm, out_hbm,
             idx_v, g_v, big0, big1, out_v,
             sem0, sem1):
    bigs = (big0, big1)
    sems = (sem0, sem1)
    wid = lax.axis_index("s") * NC + lax.axis_index("c")
    base = wid * RPW
    iota = lax.iota(jnp.int32, 16)

    pltpu.sync_copy(idx_hbm.at[pl.ds(base, RPW)], idx_v)

    def g_body(t, carry):
        v = idx_v[pl.ds(t * 16, 16)]
        g_v[pl.ds(t * 16, 16)] = lax.shift_right_logical(v, 3)
        return carry

    lax.fori_loop(0, RPW // 16, g_body, 0)

    def extract(j, big):
        def sg_body(sg, carry):
            v16 = idx_v[pl.ds(j * CHUNK + sg * 16, 16)]
            o16 = lax.shift_left(jnp.bitwise_and(v16, 7), 4)
            rvec = sg * 16 + iota
            r16 = j * CHUNK + sg * 16 + iota
            b16 = lax.shift_right_logical(r16 * 52429, 18)
            pcol = lax.shift_left(r16 - b16 * 5, 4)
            for wd in range(PAIRS):
                vals = plsc.load_gather(big, [rvec, o16 + wd])
                plsc.store_scatter(out_v, [b16, pcol + wd], vals)
            return carry

        lax.fori_loop(0, CHUNK // 16, sg_body, 0)

    copies = {}
    for j in range(NBUF):
        copies[j] = pltpu.async_copy(
            tpack_hbm.at[g_v.at[pl.ds(j * CHUNK, CHUNK)]], bigs[j], sems[j]
        )
    for j in range(CPW):
        b = j % NBUF
        copies[b].wait()
        extract(j, bigs[b])
        if j + NBUF < CPW:
            copies[b] = pltpu.async_copy(
                tpack_hbm.at[g_v.at[pl.ds((j + NBUF) * CHUNK, CHUNK)]],
                bigs[b], sems[b],
            )

    pltpu.sync_copy(out_v, out_hbm.at[pl.ds(wid * (BATCH // NW),
                                            BATCH // NW)])


_k2 = pl.kernel(
    _k2_body,
    out_type=jax.ShapeDtypeStruct((BATCH, 128), jnp.int32),
    mesh=plsc.VectorSubcoreMesh(core_axis_name="c", subcore_axis_name="s"),
    scratch_types=[
        pltpu.VMEM((RPW,), jnp.int32),
        pltpu.VMEM((RPW,), jnp.int32),
        pltpu.VMEM((CHUNK, GROUP_W), jnp.int32),
        pltpu.VMEM((CHUNK, GROUP_W), jnp.int32),
        pltpu.VMEM((BATCH // NW, 128), jnp.int32),
        pltpu.SemaphoreType.DMA,
        pltpu.SemaphoreType.DMA,
    ],
    compiler_params=pltpu.CompilerParams(
        use_tc_tiling_on_sc=True, needs_layout_passes=False
    ),
)

BLK = 1024
USEDW = NUM_WORDS * PAIRS


def _mlp_body(x_ref, w1e_ref, w1o_ref, b1_ref, w2_ref, b2_ref, o_ref):
    xi = x_ref[...]
    xe = lax.bitcast_convert_type(
        lax.shift_left(xi, jnp.int32(16)), jnp.float32)[:, :USEDW]
    xo = lax.bitcast_convert_type(
        jnp.bitwise_and(xi, jnp.int32(-65536)), jnp.float32)[:, :USEDW]
    pre = (
        jnp.dot(xe, w1e_ref[...], preferred_element_type=jnp.float32)
        + jnp.dot(xo, w1o_ref[...], preferred_element_type=jnp.float32)
        + b1_ref[...]
    )
    h = jnp.tanh(pre)
    logits = (
        jnp.dot(h, w2_ref[...], preferred_element_type=jnp.float32) + b2_ref[...]
    )
    m = jnp.max(logits, axis=-1, keepdims=True)
    s = logits - m
    o_ref[...] = s - jnp.log(jnp.sum(jnp.exp(s), axis=-1, keepdims=True))


def _mlp(x_i32, W1, b1, W2, b2):
    return pl.pallas_call(
        _mlp_body,
        grid=(BATCH // BLK,),
        in_specs=[
            pl.BlockSpec((BLK, 128), lambda i: (i, 0)),
            pl.BlockSpec((USEDW, HIDDEN), lambda i: (0, 0)),
            pl.BlockSpec((USEDW, HIDDEN), lambda i: (0, 0)),
            pl.BlockSpec((1, HIDDEN), lambda i: (0, 0)),
            pl.BlockSpec((HIDDEN, OUT), lambda i: (0, 0)),
            pl.BlockSpec((1, OUT), lambda i: (0, 0)),
        ],
        out_specs=pl.BlockSpec((BLK, OUT), lambda i: (i, 0)),
        out_shape=jax.ShapeDtypeStruct((BATCH, OUT), jnp.float32),
    )(x_i32, W1[0::2], W1[1::2], b1.reshape(1, HIDDEN), W2,
      b2.reshape(1, OUT))


def kernel(words_idxs, table, W1, b1, W2, b2):
    idx = words_idxs.astype(jnp.int32).reshape(ROWS)
    tt = table.T
    tail_bf = table[NFULL * BLKV:].astype(jnp.bfloat16)
    tail = lax.bitcast_convert_type(
        tail_bf.reshape(TAILV, PAIRS, 2), jnp.int32
    ).reshape(TAILV // GPR, GROUP_W)
    tpack = _k1(tt, tail)
    x_i32 = _k2(tpack, idx)
    return _mlp(x_i32, W1, b1, W2, b2)


# speedup vs baseline: 1.1138x; 1.1138x over previous
import jax
import jax.numpy as jnp
from jax import lax
from jax.experimental import pallas as pl
from jax.experimental.pallas import tpu as pltpu
from jax.experimental.pallas import tpu_sc as plsc

VOCAB = 1000000
EMBED = 32
NUM_WORDS = 5
HIDDEN = 256
OUT = 64
BATCH = 16384

ROWS = BATCH * NUM_WORDS
CHUNK = 128
NC = 2
NS = 16
NW = NC * NS
RPW = ROWS // NW
CPW = RPW // CHUNK
NBUF = 2

GPR = 8
PAIRS = EMBED // 2
GROUP_W = GPR * PAIRS
GROUPS = VOCAB // GPR
OUT_ROWS = ROWS * PAIRS // 128

BLKV = 512
ORB = BLKV // GPR
NFULL = VOCAB // BLKV
TAILV = VOCAB - NFULL * BLKV
BPW = NFULL // NW

_IOTA = None


def _k1_body(tt_hbm, tail_hbm, out_hbm, buf0, buf1, ob0, ob1, tob,
             sem0, sem1, osem0, osem1):
    wid = lax.axis_index("s") * NC + lax.axis_index("c")
    iota = lax.iota(jnp.int32, 16)
    nb = jnp.where(wid == 0, BPW + 1, BPW)

    def blk_of(i):
        return jnp.where(i < BPW, wid * BPW + i, NFULL - 1)

    def fire(blk, buf, sem):
        v0 = blk * BLKV
        for te in range(4):
            pltpu.async_copy(
                tt_hbm.at[pl.ds(te * 8, 8), pl.ds(v0, BLKV)],
                buf.at[pl.ds(te * 8, 8), :], sem)

    def wait_buf(buf, sem):
        pltpu.make_async_copy(tt_hbm.at[:, pl.ds(0, BLKV)], buf, sem).wait()

    def extract(buf, ob, ncol):
        def g_body(g, carry):
            c16 = g * 16 + iota
            rvec = lax.shift_right_logical(c16, 3)
            colb = lax.shift_left(jnp.bitwise_and(c16, 7), 4)
            for pw in range(PAIRS):
                va = buf[2 * pw, pl.ds(g * 16, 16)]
                vb = buf[2 * pw + 1, pl.ds(g * 16, 16)]
                packed = plsc.pack(va, vb,
                                   format=plsc.PackFormat.INTERLEAVED)
                u = plsc.bitcast(packed, jnp.int32)
                plsc.store_scatter(ob, [rvec, colb + pw], u)
            return carry

        lax.fori_loop(0, ncol // 16, g_body, 0)

    fire(blk_of(0), buf0, sem0)

    @pl.when(nb > 1)
    def _():
        fire(blk_of(1), buf1, sem1)

    def do_block(i, buf, ob, sem, osem):
        blk = blk_of(i)
        wait_buf(buf, sem)

        @pl.when(i >= 2)
        def _():
            pltpu.make_async_copy(out_hbm.at[pl.ds(0, ORB)], ob, osem).wait()
        extract(buf, ob, BLKV)

        @pl.when(i + 2 < nb)
        def _():
            fire(blk_of(i + 2), buf, sem)
        pltpu.async_copy(ob, out_hbm.at[pl.ds(blk * ORB, ORB)], osem)

    def step(i, carry):
        @pl.when(lax.rem(i, 2) == 0)
        def _():
            do_block(i, buf0, ob0, sem0, osem0)

        @pl.when(lax.rem(i, 2) == 1)
        def _():
            do_block(i, buf1, ob1, sem1, osem1)
        return carry

    lax.fori_loop(0, nb, step, 0)

    @pl.when(nb >= 1)
    def _():
        pltpu.make_async_copy(out_hbm.at[pl.ds(0, ORB)], ob0, osem0).wait()

    @pl.when(nb >= 2)
    def _():
        pltpu.make_async_copy(out_hbm.at[pl.ds(0, ORB)], ob1, osem1).wait()

    @pl.when(wid == 1)
    def _():
        pltpu.sync_copy(tail_hbm, tob)
        pltpu.sync_copy(tob, out_hbm.at[pl.ds(NFULL * ORB, TAILV // GPR)])


_k1 = pl.kernel(
    _k1_body,
    out_type=jax.ShapeDtypeStruct((GROUPS, GROUP_W), jnp.int32),
    mesh=plsc.VectorSubcoreMesh(core_axis_name="c", subcore_axis_name="s"),
    scratch_types=[
        pltpu.VMEM((EMBED, BLKV), jnp.float32),
        pltpu.VMEM((EMBED, BLKV), jnp.float32),
        pltpu.VMEM((ORB, GROUP_W), jnp.int32),
        pltpu.VMEM((ORB, GROUP_W), jnp.int32),
        pltpu.VMEM((TAILV // GPR, GROUP_W), jnp.int32),
        pltpu.SemaphoreType.DMA,
        pltpu.SemaphoreType.DMA,
        pltpu.SemaphoreType.DMA,
        pltpu.SemaphoreType.DMA,
    ],
    compiler_params=pltpu.CompilerParams(
        use_tc_tiling_on_sc=True, needs_layout_passes=False
    ),
)


def _k2_body(tpack_hbm, idx_hbm, out_hbm,
             idx_v, g_v, big0, big1, out_v,
             sem0, sem1):
    bigs = (big0, big1)
    sems = (sem0, sem1)
    wid = lax.axis_index("s") * NC + lax.axis_index("c")
    base = wid * RPW
    iota = lax.iota(jnp.int32, 16)

    pltpu.sync_copy(idx_hbm.at[pl.ds(base, RPW)], idx_v)

    def g_body(t, carry):
        v = idx_v[pl.ds(t * 16, 16)]
        g_v[pl.ds(t * 16, 16)] = lax.shift_right_logical(v, 3)
        return carry

    lax.fori_loop(0, RPW // 16, g_body, 0)

    def extract(j, big):
        def sg_body(sg, carry):
            v16 = idx_v[pl.ds(j * CHUNK + sg * 16, 16)]
            o16 = lax.shift_left(jnp.bitwise_and(v16, 7), 4)
            rvec = sg * 16 + iota
            r16 = j * CHUNK + sg * 16 + iota
            b16 = lax.shift_right_logical(r16 * 52429, 18)
            pcol = lax.shift_left(r16 - b16 * 5, 4)
            for wd in range(PAIRS):
                vals = plsc.load_gather(big, [rvec, o16 + wd])
                plsc.store_scatter(out_v, [b16, pcol + wd], vals)
            return carry

        lax.fori_loop(0, CHUNK // 16, sg_body, 0)

    copies = {}
    for j in range(NBUF):
        copies[j] = pltpu.async_copy(
            tpack_hbm.at[g_v.at[pl.ds(j * CHUNK, CHUNK)]], bigs[j], sems[j]
        )
    for j in range(CPW):
        b = j % NBUF
        copies[b].wait()
        extract(j, bigs[b])
        if j + NBUF < CPW:
            copies[b] = pltpu.async_copy(
                tpack_hbm.at[g_v.at[pl.ds((j + NBUF) * CHUNK, CHUNK)]],
                bigs[b], sems[b],
            )

    pltpu.sync_copy(out_v, out_hbm.at[pl.ds(wid * (BATCH // NW),
                                            BATCH // NW)])


_k2 = pl.kernel(
    _k2_body,
    out_type=jax.ShapeDtypeStruct((BATCH, 128), jnp.int32),
    mesh=plsc.VectorSubcoreMesh(core_axis_name="c", subcore_axis_name="s"),
    scratch_types=[
        pltpu.VMEM((RPW,), jnp.int32),
        pltpu.VMEM((RPW,), jnp.int32),
        pltpu.VMEM((CHUNK, GROUP_W), jnp.int32),
        pltpu.VMEM((CHUNK, GROUP_W), jnp.int32),
        pltpu.VMEM((BATCH // NW, 128), jnp.int32),
        pltpu.SemaphoreType.DMA,
        pltpu.SemaphoreType.DMA,
    ],
    compiler_params=pltpu.CompilerParams(
        use_tc_tiling_on_sc=True, needs_layout_passes=False
    ),
)

BLK = 1024
USEDW = NUM_WORDS * PAIRS


def _mlp_body(x_ref, w1e_ref, w1o_ref, b1_ref, w2_ref, b2_ref, o_ref):
    xi = x_ref[...]
    xe = lax.bitcast_convert_type(
        lax.shift_left(xi, jnp.int32(16)), jnp.float32)[:, :USEDW]
    xo = lax.bitcast_convert_type(
        jnp.bitwise_and(xi, jnp.int32(-65536)), jnp.float32)[:, :USEDW]
    pre = (
        jnp.dot(xe, w1e_ref[...], preferred_element_type=jnp.float32)
        + jnp.dot(xo, w1o_ref[...], preferred_element_type=jnp.float32)
        + b1_ref[...]
    )
    h = jnp.tanh(pre)
    logits = (
        jnp.dot(h, w2_ref[...], preferred_element_type=jnp.float32) + b2_ref[...]
    )
    m = jnp.max(logits, axis=-1, keepdims=True)
    s = logits - m
    o_ref[...] = s - jnp.log(jnp.sum(jnp.exp(s), axis=-1, keepdims=True))


def _mlp(x_i32, W1, b1, W2, b2):
    return pl.pallas_call(
        _mlp_body,
        grid=(BATCH // BLK,),
        in_specs=[
            pl.BlockSpec((BLK, 128), lambda i: (i, 0)),
            pl.BlockSpec((USEDW, HIDDEN), lambda i: (0, 0)),
            pl.BlockSpec((USEDW, HIDDEN), lambda i: (0, 0)),
            pl.BlockSpec((1, HIDDEN), lambda i: (0, 0)),
            pl.BlockSpec((HIDDEN, OUT), lambda i: (0, 0)),
            pl.BlockSpec((1, OUT), lambda i: (0, 0)),
        ],
        out_specs=pl.BlockSpec((BLK, OUT), lambda i: (i, 0)),
        out_shape=jax.ShapeDtypeStruct((BATCH, OUT), jnp.float32),
    )(x_i32, W1[0::2], W1[1::2], b1.reshape(1, HIDDEN), W2,
      b2.reshape(1, OUT))


def kernel(words_idxs, table, W1, b1, W2, b2):
    idx = words_idxs.astype(jnp.int32).reshape(ROWS)
    tt = table.T
    tail_bf = table[NFULL * BLKV:].astype(jnp.bfloat16)
    tail = lax.bitcast_convert_type(
        tail_bf.reshape(TAILV, PAIRS, 2), jnp.int32
    ).reshape(TAILV // GPR, GROUP_W)
    tpack = _k1(tt, tail)
    x_i32 = _k2(tpack, idx)
    return _mlp(x_i32, W1, b1, W2, b2)
